# baseline (speedup 1.0000x reference)
import functools

import jax
import jax.numpy as jnp
from jax import lax
from jax.experimental import pallas as pl
from jax.experimental.pallas import tpu as pltpu
from jax.experimental.pallas import tpu_sc as plsc

LAT = 128
NNODE = 10000
NEDGE = 320000
EB = 3200
NC = 2
NW = 32
CH = 40
Q = 2

_MESH = plsc.VectorSubcoreMesh(core_axis_name="c", subcore_axis_name="s")


def _ln(h, g, b):
    m = jnp.mean(h, axis=-1, keepdims=True)
    v = jnp.mean((h - m) ** 2, axis=-1, keepdims=True)
    return (h - m) * jax.lax.rsqrt(v + 1e-5) * g + b


def _dot(a, b):
    return jnp.dot(a, b, preferred_element_type=jnp.float32)


def _sc_gather_pair(nl, col3, row3):
    _, nch, ch = col3.shape
    epw = nch * ch
    ne = NW * epw

    @functools.partial(
        pl.kernel,
        mesh=_MESH,
        out_type=[jax.ShapeDtypeStruct((ne, LAT), jnp.float32)] * 2,
        scratch_types=[
            pltpu.VMEM((nch, ch), jnp.int32),
            pltpu.VMEM((nch, ch), jnp.int32),
            pltpu.VMEM((2, ch, LAT), jnp.float32),
            pltpu.VMEM((2, ch, LAT), jnp.float32),
            pltpu.SemaphoreType.DMA,
            pltpu.SemaphoreType.DMA,
            pltpu.SemaphoreType.DMA,
            pltpu.SemaphoreType.DMA,
        ],
    )
    def k(nl_h, col_h, row_h, outc_h, outr_h, idxc, idxr, bufc, bufr,
          sc0, sc1, sr0, sr1):
        wid = lax.axis_index("s") * NC + lax.axis_index("c")
        base = wid * epw
        pltpu.sync_copy(col_h.at[wid], idxc)
        pltpu.sync_copy(row_h.at[wid], idxr)
        csems = (sc0, sc1)
        rsems = (sr0, sr1)

        def start(j, b):
            pltpu.async_copy(nl_h.at[idxc.at[j]], bufc.at[b], csems[b])
            pltpu.async_copy(nl_h.at[idxr.at[j]], bufr.at[b], rsems[b])

        def finish(j, b):
            pltpu.make_async_copy(nl_h.at[idxc.at[j]], bufc.at[b],
                                  csems[b]).wait()
            pltpu.make_async_copy(nl_h.at[idxr.at[j]], bufr.at[b],
                                  rsems[b]).wait()
            pltpu.sync_copy(bufc.at[b], outc_h.at[pl.ds(base + j * ch, ch)])
            pltpu.sync_copy(bufr.at[b], outr_h.at[pl.ds(base + j * ch, ch)])

        start(0, 0)
        start(1, 1)

        def body(i, carry):
            j0 = i * 2
            for b in range(2):
                j = j0 + b
                finish(j, b)

                @pl.when(j + 2 < nch)
                def _():
                    start(j + 2, b)
            return carry

        lax.fori_loop(0, nch // 2, body, 0)
        if nch % 2 == 1:
            finish(nch - 1, 0)

    return k(nl, col3, row3)


def _sc_scatter(msg, col3, zrows):
    _, nch, ch = col3.shape
    epw = nch * ch

    @functools.partial(
        pl.kernel,
        mesh=_MESH,
        out_type=jax.ShapeDtypeStruct((NC, NNODE, LAT), jnp.float32),
        scratch_types=[
            pltpu.VMEM((nch, ch), jnp.int32),
            pltpu.VMEM((2, ch, LAT), jnp.float32),
            pltpu.VMEM_SHARED((NNODE, LAT), jnp.float32),
            pltpu.SemaphoreType.DMA,
            pltpu.SemaphoreType.DMA,
        ],
    )
    def k(msg_h, col_h, z_h, out_h, idxc, mbuf, acc, s0, s1):
        sid = lax.axis_index("s")
        cid = lax.axis_index("c")
        wid = sid * NC + cid
        base = wid * epw
        pltpu.sync_copy(col_h.at[wid], idxc)

        nzch = NNODE // CH

        def zbody(t, carry):
            j = t * 16 + sid

            @pl.when(j < nzch)
            def _():
                pltpu.sync_copy(z_h, acc.at[pl.ds(j * CH, CH)])
            return carry

        lax.fori_loop(0, (nzch + 15) // 16, zbody, 0)
        plsc.subcore_barrier()

        sems = (s0, s1)

        def load(j, b):
            pltpu.async_copy(msg_h.at[pl.ds(base + j * ch, ch)], mbuf.at[b],
                             sems[b])

        def finish(j, b):
            pltpu.make_async_copy(msg_h.at[pl.ds(base + j * ch, ch)],
                                  mbuf.at[b], sems[b]).wait()
            pltpu.sync_copy(mbuf.at[b], acc.at[idxc.at[j]], add=True)

        load(0, 0)
        load(1, 1)

        def body(i, carry):
            j0 = i * 2
            for b in range(2):
                j = j0 + b
                finish(j, b)

                @pl.when(j + 2 < nch)
                def _():
                    load(j + 2, b)
            return carry

        lax.fori_loop(0, nch // 2, body, 0)
        if nch % 2 == 1:
            finish(nch - 1, 0)
        plsc.subcore_barrier()

        def wbody(t, carry):
            j = t * 16 + sid

            @pl.when(j < nzch)
            def _():
                pltpu.sync_copy(acc.at[pl.ds(j * CH, CH)],
                                out_h.at[cid].at[pl.ds(j * CH, CH)])
            return carry

        lax.fori_loop(0, (nzch + 15) // 16, wbody, 0)

    return k(msg, col3, zrows)


def _encode_nodes_body(x, neW1t, neb1, neW2t, neb2, neg, nebe, nl_o):
    h = jax.nn.relu(_dot(x[...], neW1t[...]) + neb1[...])
    h = jax.nn.relu(_dot(h, neW2t[...]) + neb2[...])
    nl_o[...] = _ln(h, neg[...], nebe[...])


def _encode_nodes(x, neW1t, neb1, neW2t, neb2, neg, nebe):
    return pl.pallas_call(
        _encode_nodes_body,
        out_shape=jax.ShapeDtypeStruct((NNODE, LAT), jnp.float32),
    )(x, neW1t, neb1, neW2t, neb2, neg, nebe)


def _edge1_body(ea, nlc, nlr, eeW1t, eeb1, eeW2t, eeb2, eeg, eebe,
                w1a, w1b, w1ct, b1, w2t, b2, gg, gbe, msg_o, el_o):
    h = jax.nn.relu(_dot(ea[...], eeW1t[...]) + eeb1[...])
    h = jax.nn.relu(_dot(h, eeW2t[...]) + eeb2[...])
    el = _ln(h, eeg[...], eebe[...])
    ce = _dot(el, w1ct[...]) + b1[...]
    tc = _dot(nlc[...], w1a[...])
    tr = _dot(nlr[...], w1b[...])
    m = jax.nn.relu(tc + tr + ce)
    m = jax.nn.relu(_dot(m, w2t[...]) + b2[...])
    msg_o[...] = _ln(m, gg[...], gbe[...])
    n = jax.nn.relu(_dot(nlr[...], w1a[...]) + _dot(nlc[...], w1b[...]) + ce)
    n = jax.nn.relu(_dot(n, w2t[...]) + b2[...])
    el_o[...] = _ln(n, gg[...], gbe[...]) + el


def _edge_step1(ea, nlc, nlr, eeW1t, eeb1, eeW2t, eeb2, eeg, eebe,
                w1a, w1b, w1ct, b1, w2t, b2, gg, gbe):
    ne = ea.shape[0]
    nblk = ne // EB
    eb_spec = pl.BlockSpec((EB, LAT), lambda i: (i, 0))
    ea_spec = pl.BlockSpec((EB, 16), lambda i: (i, 0))
    w_spec = lambda s: pl.BlockSpec(s, lambda i: (0,) * len(s))
    return pl.pallas_call(
        _edge1_body,
        grid=(nblk,),
        in_specs=[ea_spec, eb_spec, eb_spec,
                  w_spec((16, LAT)), w_spec((1, LAT)), w_spec((LAT, LAT)),
                  w_spec((1, LAT)), w_spec((1, LAT)), w_spec((1, LAT)),
                  w_spec((LAT, LAT)), w_spec((LAT, LAT)), w_spec((LAT, LAT)),
                  w_spec((1, LAT)), w_spec((LAT, LAT)), w_spec((1, LAT)),
                  w_spec((1, LAT)), w_spec((1, LAT))],
        out_specs=[eb_spec, eb_spec],
        out_shape=[jax.ShapeDtypeStruct((ne, LAT), jnp.float32)] * 2,
    )(ea, nlc, nlr, eeW1t, eeb1, eeW2t, eeb2, eeg, eebe,
      w1a, w1b, w1ct, b1, w2t, b2, gg, gbe)


def _edge2_body(el, nlc, nlr, w1a, w1b, w1ct, b1, w2t, b2, gg, gbe, msg_o):
    ce = _dot(el[...], w1ct[...]) + b1[...]
    m = jax.nn.relu(_dot(nlc[...], w1a[...]) + _dot(nlr[...], w1b[...]) + ce)
    m = jax.nn.relu(_dot(m, w2t[...]) + b2[...])
    msg_o[...] = _ln(m, gg[...], gbe[...])


def _edge_step2(el, nlc, nlr, w1a, w1b, w1ct, b1, w2t, b2, gg, gbe):
    ne = el.shape[0]
    nblk = ne // EB
    eb_spec = pl.BlockSpec((EB, LAT), lambda i: (i, 0))
    w_spec = lambda s: pl.BlockSpec(s, lambda i: (0,) * len(s))
    return pl.pallas_call(
        _edge2_body,
        grid=(nblk,),
        in_specs=[eb_spec, eb_spec, eb_spec,
                  w_spec((LAT, LAT)), w_spec((LAT, LAT)), w_spec((LAT, LAT)),
                  w_spec((1, LAT)), w_spec((LAT, LAT)), w_spec((1, LAT)),
                  w_spec((1, LAT)), w_spec((1, LAT))],
        out_specs=eb_spec,
        out_shape=jax.ShapeDtypeStruct((ne, LAT), jnp.float32),
    )(el, nlc, nlr, w1a, w1b, w1ct, b1, w2t, b2, gg, gbe)


def _node_upd_body(ps, nl, wat, wbt, b1, w2t, b2, gg, gbe, nl_o):
    aggr = ps[0]
    for i in range(1, ps.shape[0]):
        aggr = aggr + ps[i]
    h = jax.nn.relu(_dot(aggr, wat[...]) + _dot(nl[...], wbt[...]) + b1[...])
    h = jax.nn.relu(_dot(h, w2t[...]) + b2[...])
    nl_o[...] = _ln(h, gg[...], gbe[...]) + nl[...]


def _node_update(ps, nl, wat, wbt, b1, w2t, b2, gg, gbe):
    return pl.pallas_call(
        _node_upd_body,
        out_shape=jax.ShapeDtypeStruct((NNODE, LAT), jnp.float32),
    )(ps, nl, wat, wbt, b1, w2t, b2, gg, gbe)


def _decode_body(ps, nl, wat, wbt, b1, w2t, b2, gg, gbe,
                 g1wt, g1bi, g1hr, g1hz, g1hn,
                 g2wt, g2bi, g2hr, g2hz, g2hn,
                 dW1t, db1, dW2t, db2, out_o):
    aggr = ps[0]
    for i in range(1, ps.shape[0]):
        aggr = aggr + ps[i]
    h = jax.nn.relu(_dot(aggr, wat[...]) + _dot(nl[...], wbt[...]) + b1[...])
    h = jax.nn.relu(_dot(h, w2t[...]) + b2[...])
    nl2 = _ln(h, gg[...], gbe[...]) + nl[...]

    gi = _dot(nl2, g1wt[...]) + g1bi[...]
    r = jax.nn.sigmoid(gi[:, :LAT] + g1hr[...])
    z = jax.nn.sigmoid(gi[:, LAT:2 * LAT] + g1hz[...])
    nn = jnp.tanh(gi[:, 2 * LAT:] + r * g1hn[...])
    h1 = (1.0 - z) * nn

    gi2 = _dot(h1, g2wt[...]) + g2bi[...]
    r2 = jax.nn.sigmoid(gi2[:, :LAT] + g2hr[...])
    z2 = jax.nn.sigmoid(gi2[:, LAT:2 * LAT] + g2hz[...])
    nn2 = jnp.tanh(gi2[:, 2 * LAT:] + r2 * g2hn[...])
    h2 = (1.0 - z2) * nn2

    d = jax.nn.relu(_dot(h2, dW1t[...]) + db1[...])
    out_o[...] = _dot(d, dW2t[...]) + db2[...]


def _decode(ps, nl, wat, wbt, b1, w2t, b2, gg, gbe,
            g1wt, g1bi, g1hr, g1hz, g1hn, g2wt, g2bi, g2hr, g2hz, g2hn,
            dW1t, db1, dW2t, db2):
    return pl.pallas_call(
        _decode_body,
        out_shape=jax.ShapeDtypeStruct((NNODE, LAT), jnp.float32),
    )(ps, nl, wat, wbt, b1, w2t, b2, gg, gbe,
      g1wt, g1bi, g1hr, g1hz, g1hn, g2wt, g2bi, g2hr, g2hz, g2hn,
      dW1t, db1, dW2t, db2)


def kernel(x, edge_index, edge_attr, ne_W1, ne_b1, ne_W2, ne_b2, ne_g, ne_be,
           ee_W1, ee_b1, ee_W2, ee_b2, ee_g, ee_be,
           gbe_W1, gbe_b1, gbe_W2, gbe_b2, gbe_g, gbe_be,
           gbn_W1, gbn_b1, gbn_W2, gbn_b2, gbn_g, gbn_be,
           g1_Wih, g1_Whh, g1_bih, g1_bhh,
           g2_Wih, g2_Whh, g2_bih, g2_bhh,
           dec_W1, dec_b1, dec_W2, dec_b2):
    eq = NEDGE // Q
    nchq = eq // (NW * CH)
    row4 = edge_index[0].reshape(Q, NW, nchq, CH)
    col4 = edge_index[1].reshape(Q, NW, nchq, CH)
    ea3 = edge_attr.reshape(Q, eq, 16)
    r2 = lambda v: v.reshape(1, -1)
    zrows = jnp.zeros((CH, LAT), jnp.float32)

    w1at = gbe_W1[:, :LAT].T
    w1bt = gbe_W1[:, LAT:2 * LAT].T
    w1ct = gbe_W1[:, 2 * LAT:].T
    w2te = gbe_W2.T
    wat = gbn_W1[:, :LAT].T
    wbt = gbn_W1[:, LAT:].T

    nl = _encode_nodes(x, ne_W1.T, r2(ne_b1), ne_W2.T, r2(ne_b2),
                       r2(ne_g), r2(ne_be))

    gpairs = [_sc_gather_pair(nl, col4[q], row4[q]) for q in range(Q)]
    msgs, els = [], []
    for q in range(Q):
        nlc, nlr = gpairs[q]
        m_q, el_q = _edge_step1(ea3[q], nlc, nlr, ee_W1.T, r2(ee_b1),
                                ee_W2.T, r2(ee_b2), r2(ee_g), r2(ee_be),
                                w1at, w1bt, w1ct, r2(gbe_b1), w2te,
                                r2(gbe_b2), r2(gbe_g), r2(gbe_be))
        msgs.append(m_q)
        els.append(el_q)
    parts = [_sc_scatter(msgs[q], col4[q], zrows) for q in range(Q)]
    ps = jnp.concatenate(parts, axis=0)
    nl = _node_update(ps, nl, wat, wbt, r2(gbn_b1), gbn_W2.T,
                      r2(gbn_b2), r2(gbn_g), r2(gbn_be))

    gpairs = [_sc_gather_pair(nl, col4[q], row4[q]) for q in range(Q)]
    msgs = []
    for q in range(Q):
        nlc, nlr = gpairs[q]
        msgs.append(_edge_step2(els[q], nlc, nlr, w1at, w1bt, w1ct,
                                r2(gbe_b1), w2te, r2(gbe_b2), r2(gbe_g),
                                r2(gbe_be)))
    parts = [_sc_scatter(msgs[q], col4[q], zrows) for q in range(Q)]
    ps = jnp.concatenate(parts, axis=0)

    g1hr, g1hz, g1hn = g1_bhh[:LAT], g1_bhh[LAT:2 * LAT], g1_bhh[2 * LAT:]
    g2hr, g2hz, g2hn = g2_bhh[:LAT], g2_bhh[LAT:2 * LAT], g2_bhh[2 * LAT:]
    dW2t = jnp.zeros((LAT, LAT), jnp.float32).at[:, :3].set(dec_W2.T)
    db2 = jnp.zeros((1, LAT), jnp.float32).at[:, :3].set(dec_b2)
    dec = _decode(ps, nl, wat, wbt, r2(gbn_b1), gbn_W2.T, r2(gbn_b2),
                  r2(gbn_g), r2(gbn_be),
                  g1_Wih.T, r2(g1_bih), r2(g1hr), r2(g1hz), r2(g1hn),
                  g2_Wih.T, r2(g2_bih), r2(g2hr), r2(g2hz), r2(g2hn),
                  dec_W1.T, r2(dec_b1), dW2t, db2)
    return dec[None, :, :3]


# speedup vs baseline: 1.0918x; 1.0918x over previous
import functools

import jax
import jax.numpy as jnp
from jax import lax
from jax.experimental import pallas as pl
from jax.experimental.pallas import tpu as pltpu
from jax.experimental.pallas import tpu_sc as plsc

LAT = 128
NNODE = 10000
NEDGE = 320000
EB = 3200
NC = 2
NW = 32
CH = 80
Q = 1
ND = 4
NDS = 3

_MESH = plsc.VectorSubcoreMesh(core_axis_name="c", subcore_axis_name="s")


def _ln(h, g, b):
    m = jnp.mean(h, axis=-1, keepdims=True)
    v = jnp.mean((h - m) ** 2, axis=-1, keepdims=True)
    return (h - m) * jax.lax.rsqrt(v + 1e-5) * g + b


def _dot(a, b):
    return jnp.dot(a, b, preferred_element_type=jnp.float32)


def _sc_gather_pair(nl, col3, row3):
    _, nch, ch = col3.shape
    epw = nch * ch
    ne = NW * epw

    @functools.partial(
        pl.kernel,
        mesh=_MESH,
        out_type=[jax.ShapeDtypeStruct((ne, LAT), jnp.float32)] * 2,
        scratch_types=[
            pltpu.VMEM((nch, ch), jnp.int32),
            pltpu.VMEM((nch, ch), jnp.int32),
            pltpu.VMEM((ND, ch, LAT), jnp.float32),
            pltpu.VMEM((ND, ch, LAT), jnp.float32),
        ] + [pltpu.SemaphoreType.DMA] * (2 * ND),
    )
    def k(nl_h, col_h, row_h, outc_h, outr_h, idxc, idxr, bufc, bufr, *sems):
        wid = lax.axis_index("s") * NC + lax.axis_index("c")
        base = wid * epw
        pltpu.sync_copy(col_h.at[wid], idxc)
        pltpu.sync_copy(row_h.at[wid], idxr)
        csems = sems[:ND]
        rsems = sems[ND:]

        def start(j, b):
            pltpu.async_copy(nl_h.at[idxc.at[j]], bufc.at[b], csems[b])
            pltpu.async_copy(nl_h.at[idxr.at[j]], bufr.at[b], rsems[b])

        def finish(j, b):
            pltpu.make_async_copy(nl_h.at[idxc.at[j]], bufc.at[b],
                                  csems[b]).wait()
            pltpu.make_async_copy(nl_h.at[idxr.at[j]], bufr.at[b],
                                  rsems[b]).wait()
            pltpu.sync_copy(bufc.at[b], outc_h.at[pl.ds(base + j * ch, ch)])
            pltpu.sync_copy(bufr.at[b], outr_h.at[pl.ds(base + j * ch, ch)])

        for b in range(ND):
            start(b, b)

        def body(i, carry):
            j0 = i * ND
            for b in range(ND):
                j = j0 + b
                finish(j, b)

                @pl.when(j + ND < nch)
                def _():
                    start(j + ND, b)
            return carry

        lax.fori_loop(0, nch // ND, body, 0)
        for j in range((nch // ND) * ND, nch):
            finish(j, j % ND)

    return k(nl, col3, row3)


def _sc_scatter(msg, col3, zrows):
    _, nch, ch = col3.shape
    epw = nch * ch

    @functools.partial(
        pl.kernel,
        mesh=_MESH,
        out_type=jax.ShapeDtypeStruct((NC, NNODE, LAT), jnp.float32),
        scratch_types=[
            pltpu.VMEM((nch, ch), jnp.int32),
            pltpu.VMEM((NDS, ch, LAT), jnp.float32),
            pltpu.VMEM_SHARED((NNODE, LAT), jnp.float32),
        ] + [pltpu.SemaphoreType.DMA] * NDS,
    )
    def k(msg_h, col_h, z_h, out_h, idxc, mbuf, acc, *sems):
        sid = lax.axis_index("s")
        cid = lax.axis_index("c")
        wid = sid * NC + cid
        base = wid * epw
        pltpu.sync_copy(col_h.at[wid], idxc)

        nzch = NNODE // CH

        def zbody(t, carry):
            j = t * 16 + sid

            @pl.when(j < nzch)
            def _():
                pltpu.sync_copy(z_h, acc.at[pl.ds(j * CH, CH)])
            return carry

        lax.fori_loop(0, (nzch + 15) // 16, zbody, 0)
        plsc.subcore_barrier()

        def load(j, b):
            pltpu.async_copy(msg_h.at[pl.ds(base + j * ch, ch)], mbuf.at[b],
                             sems[b])

        def finish(j, b):
            pltpu.make_async_copy(msg_h.at[pl.ds(base + j * ch, ch)],
                                  mbuf.at[b], sems[b]).wait()
            pltpu.sync_copy(mbuf.at[b], acc.at[idxc.at[j]], add=True)

        for b in range(NDS):
            load(b, b)

        def body(i, carry):
            j0 = i * NDS
            for b in range(NDS):
                j = j0 + b
                finish(j, b)

                @pl.when(j + NDS < nch)
                def _():
                    load(j + NDS, b)
            return carry

        lax.fori_loop(0, nch // NDS, body, 0)
        for j in range((nch // NDS) * NDS, nch):
            finish(j, j % NDS)
        plsc.subcore_barrier()

        def wbody(t, carry):
            j = t * 16 + sid

            @pl.when(j < nzch)
            def _():
                pltpu.sync_copy(acc.at[pl.ds(j * CH, CH)],
                                out_h.at[cid].at[pl.ds(j * CH, CH)])
            return carry

        lax.fori_loop(0, (nzch + 15) // 16, wbody, 0)

    return k(msg, col3, zrows)


def _encode_nodes_body(x, neW1t, neb1, neW2t, neb2, neg, nebe, nl_o):
    h = jax.nn.relu(_dot(x[...], neW1t[...]) + neb1[...])
    h = jax.nn.relu(_dot(h, neW2t[...]) + neb2[...])
    nl_o[...] = _ln(h, neg[...], nebe[...])


def _encode_nodes(x, neW1t, neb1, neW2t, neb2, neg, nebe):
    return pl.pallas_call(
        _encode_nodes_body,
        out_shape=jax.ShapeDtypeStruct((NNODE, LAT), jnp.float32),
    )(x, neW1t, neb1, neW2t, neb2, neg, nebe)


def _edge1_body(ea, nlc, nlr, eeW1t, eeb1, eeW2t, eeb2, eeg, eebe,
                w1a, w1b, w1ct, b1, w2t, b2, gg, gbe, msg_o, el_o):
    h = jax.nn.relu(_dot(ea[...], eeW1t[...]) + eeb1[...])
    h = jax.nn.relu(_dot(h, eeW2t[...]) + eeb2[...])
    el = _ln(h, eeg[...], eebe[...])
    ce = _dot(el, w1ct[...]) + b1[...]
    tc = _dot(nlc[...], w1a[...])
    tr = _dot(nlr[...], w1b[...])
    m = jax.nn.relu(tc + tr + ce)
    m = jax.nn.relu(_dot(m, w2t[...]) + b2[...])
    msg_o[...] = _ln(m, gg[...], gbe[...])
    n = jax.nn.relu(_dot(nlr[...], w1a[...]) + _dot(nlc[...], w1b[...]) + ce)
    n = jax.nn.relu(_dot(n, w2t[...]) + b2[...])
    el_o[...] = _ln(n, gg[...], gbe[...]) + el


def _edge_step1(ea, nlc, nlr, eeW1t, eeb1, eeW2t, eeb2, eeg, eebe,
                w1a, w1b, w1ct, b1, w2t, b2, gg, gbe):
    ne = ea.shape[0]
    nblk = ne // EB
    eb_spec = pl.BlockSpec((EB, LAT), lambda i: (i, 0))
    ea_spec = pl.BlockSpec((EB, 16), lambda i: (i, 0))
    w_spec = lambda s: pl.BlockSpec(s, lambda i: (0,) * len(s))
    return pl.pallas_call(
        _edge1_body,
        grid=(nblk,),
        in_specs=[ea_spec, eb_spec, eb_spec,
                  w_spec((16, LAT)), w_spec((1, LAT)), w_spec((LAT, LAT)),
                  w_spec((1, LAT)), w_spec((1, LAT)), w_spec((1, LAT)),
                  w_spec((LAT, LAT)), w_spec((LAT, LAT)), w_spec((LAT, LAT)),
                  w_spec((1, LAT)), w_spec((LAT, LAT)), w_spec((1, LAT)),
                  w_spec((1, LAT)), w_spec((1, LAT))],
        out_specs=[eb_spec, eb_spec],
        out_shape=[jax.ShapeDtypeStruct((ne, LAT), jnp.float32)] * 2,
    )(ea, nlc, nlr, eeW1t, eeb1, eeW2t, eeb2, eeg, eebe,
      w1a, w1b, w1ct, b1, w2t, b2, gg, gbe)


def _edge2_body(el, nlc, nlr, w1a, w1b, w1ct, b1, w2t, b2, gg, gbe, msg_o):
    ce = _dot(el[...], w1ct[...]) + b1[...]
    m = jax.nn.relu(_dot(nlc[...], w1a[...]) + _dot(nlr[...], w1b[...]) + ce)
    m = jax.nn.relu(_dot(m, w2t[...]) + b2[...])
    msg_o[...] = _ln(m, gg[...], gbe[...])


def _edge_step2(el, nlc, nlr, w1a, w1b, w1ct, b1, w2t, b2, gg, gbe):
    ne = el.shape[0]
    nblk = ne // EB
    eb_spec = pl.BlockSpec((EB, LAT), lambda i: (i, 0))
    w_spec = lambda s: pl.BlockSpec(s, lambda i: (0,) * len(s))
    return pl.pallas_call(
        _edge2_body,
        grid=(nblk,),
        in_specs=[eb_spec, eb_spec, eb_spec,
                  w_spec((LAT, LAT)), w_spec((LAT, LAT)), w_spec((LAT, LAT)),
                  w_spec((1, LAT)), w_spec((LAT, LAT)), w_spec((1, LAT)),
                  w_spec((1, LAT)), w_spec((1, LAT))],
        out_specs=eb_spec,
        out_shape=jax.ShapeDtypeStruct((ne, LAT), jnp.float32),
    )(el, nlc, nlr, w1a, w1b, w1ct, b1, w2t, b2, gg, gbe)


def _node_upd_body(ps, nl, wat, wbt, b1, w2t, b2, gg, gbe, nl_o):
    aggr = ps[0]
    for i in range(1, ps.shape[0]):
        aggr = aggr + ps[i]
    h = jax.nn.relu(_dot(aggr, wat[...]) + _dot(nl[...], wbt[...]) + b1[...])
    h = jax.nn.relu(_dot(h, w2t[...]) + b2[...])
    nl_o[...] = _ln(h, gg[...], gbe[...]) + nl[...]


def _node_update(ps, nl, wat, wbt, b1, w2t, b2, gg, gbe):
    return pl.pallas_call(
        _node_upd_body,
        out_shape=jax.ShapeDtypeStruct((NNODE, LAT), jnp.float32),
    )(ps, nl, wat, wbt, b1, w2t, b2, gg, gbe)


def _decode_body(ps, nl, wat, wbt, b1, w2t, b2, gg, gbe,
                 g1wt, g1bi, g1hr, g1hz, g1hn,
                 g2wt, g2bi, g2hr, g2hz, g2hn,
                 dW1t, db1, dW2t, db2, out_o):
    aggr = ps[0]
    for i in range(1, ps.shape[0]):
        aggr = aggr + ps[i]
    h = jax.nn.relu(_dot(aggr, wat[...]) + _dot(nl[...], wbt[...]) + b1[...])
    h = jax.nn.relu(_dot(h, w2t[...]) + b2[...])
    nl2 = _ln(h, gg[...], gbe[...]) + nl[...]

    gi = _dot(nl2, g1wt[...]) + g1bi[...]
    r = jax.nn.sigmoid(gi[:, :LAT] + g1hr[...])
    z = jax.nn.sigmoid(gi[:, LAT:2 * LAT] + g1hz[...])
    nn = jnp.tanh(gi[:, 2 * LAT:] + r * g1hn[...])
    h1 = (1.0 - z) * nn

    gi2 = _dot(h1, g2wt[...]) + g2bi[...]
    r2 = jax.nn.sigmoid(gi2[:, :LAT] + g2hr[...])
    z2 = jax.nn.sigmoid(gi2[:, LAT:2 * LAT] + g2hz[...])
    nn2 = jnp.tanh(gi2[:, 2 * LAT:] + r2 * g2hn[...])
    h2 = (1.0 - z2) * nn2

    d = jax.nn.relu(_dot(h2, dW1t[...]) + db1[...])
    out_o[...] = _dot(d, dW2t[...]) + db2[...]


def _decode(ps, nl, wat, wbt, b1, w2t, b2, gg, gbe,
            g1wt, g1bi, g1hr, g1hz, g1hn, g2wt, g2bi, g2hr, g2hz, g2hn,
            dW1t, db1, dW2t, db2):
    return pl.pallas_call(
        _decode_body,
        out_shape=jax.ShapeDtypeStruct((NNODE, LAT), jnp.float32),
    )(ps, nl, wat, wbt, b1, w2t, b2, gg, gbe,
      g1wt, g1bi, g1hr, g1hz, g1hn, g2wt, g2bi, g2hr, g2hz, g2hn,
      dW1t, db1, dW2t, db2)


def kernel(x, edge_index, edge_attr, ne_W1, ne_b1, ne_W2, ne_b2, ne_g, ne_be,
           ee_W1, ee_b1, ee_W2, ee_b2, ee_g, ee_be,
           gbe_W1, gbe_b1, gbe_W2, gbe_b2, gbe_g, gbe_be,
           gbn_W1, gbn_b1, gbn_W2, gbn_b2, gbn_g, gbn_be,
           g1_Wih, g1_Whh, g1_bih, g1_bhh,
           g2_Wih, g2_Whh, g2_bih, g2_bhh,
           dec_W1, dec_b1, dec_W2, dec_b2):
    eq = NEDGE // Q
    nchq = eq // (NW * CH)
    row4 = edge_index[0].reshape(Q, NW, nchq, CH)
    col4 = edge_index[1].reshape(Q, NW, nchq, CH)
    ea3 = edge_attr.reshape(Q, eq, 16)
    r2 = lambda v: v.reshape(1, -1)
    zrows = jnp.zeros((CH, LAT), jnp.float32)

    w1at = gbe_W1[:, :LAT].T
    w1bt = gbe_W1[:, LAT:2 * LAT].T
    w1ct = gbe_W1[:, 2 * LAT:].T
    w2te = gbe_W2.T
    wat = gbn_W1[:, :LAT].T
    wbt = gbn_W1[:, LAT:].T

    nl = _encode_nodes(x, ne_W1.T, r2(ne_b1), ne_W2.T, r2(ne_b2),
                       r2(ne_g), r2(ne_be))

    gpairs = [_sc_gather_pair(nl, col4[q], row4[q]) for q in range(Q)]
    msgs, els = [], []
    for q in range(Q):
        nlc, nlr = gpairs[q]
        m_q, el_q = _edge_step1(ea3[q], nlc, nlr, ee_W1.T, r2(ee_b1),
                                ee_W2.T, r2(ee_b2), r2(ee_g), r2(ee_be),
                                w1at, w1bt, w1ct, r2(gbe_b1), w2te,
                                r2(gbe_b2), r2(gbe_g), r2(gbe_be))
        msgs.append(m_q)
        els.append(el_q)
    parts = [_sc_scatter(msgs[q], col4[q], zrows) for q in range(Q)]
    ps = jnp.concatenate(parts, axis=0)
    nl = _node_update(ps, nl, wat, wbt, r2(gbn_b1), gbn_W2.T,
                      r2(gbn_b2), r2(gbn_g), r2(gbn_be))

    gpairs = [_sc_gather_pair(nl, col4[q], row4[q]) for q in range(Q)]
    msgs = []
    for q in range(Q):
        nlc, nlr = gpairs[q]
        msgs.append(_edge_step2(els[q], nlc, nlr, w1at, w1bt, w1ct,
                                r2(gbe_b1), w2te, r2(gbe_b2), r2(gbe_g),
                                r2(gbe_be)))
    parts = [_sc_scatter(msgs[q], col4[q], zrows) for q in range(Q)]
    ps = jnp.concatenate(parts, axis=0)

    g1hr, g1hz, g1hn = g1_bhh[:LAT], g1_bhh[LAT:2 * LAT], g1_bhh[2 * LAT:]
    g2hr, g2hz, g2hn = g2_bhh[:LAT], g2_bhh[LAT:2 * LAT], g2_bhh[2 * LAT:]
    dW2t = jnp.zeros((LAT, LAT), jnp.float32).at[:, :3].set(dec_W2.T)
    db2 = jnp.zeros((1, LAT), jnp.float32).at[:, :3].set(dec_b2)
    dec = _decode(ps, nl, wat, wbt, r2(gbn_b1), gbn_W2.T, r2(gbn_b2),
                  r2(gbn_g), r2(gbn_be),
                  g1_Wih.T, r2(g1_bih), r2(g1hr), r2(g1hz), r2(g1hn),
                  g2_Wih.T, r2(g2_bih), r2(g2hr), r2(g2hz), r2(g2hn),
                  dec_W1.T, r2(dec_b1), dW2t, db2)
    return dec[None, :, :3]


# speedup vs baseline: 1.1105x; 1.0171x over previous
import functools

import jax
import jax.numpy as jnp
from jax import lax
from jax.experimental import pallas as pl
from jax.experimental.pallas import tpu as pltpu
from jax.experimental.pallas import tpu_sc as plsc

LAT = 128
NNODE = 10000
NEDGE = 320000
EB = 6400
NC = 2
NW = 32
CH = 80
Q = 1
ND = 4
NDS = 3

_MESH = plsc.VectorSubcoreMesh(core_axis_name="c", subcore_axis_name="s")


def _ln(h, g, b):
    m = jnp.mean(h, axis=-1, keepdims=True)
    v = jnp.mean((h - m) ** 2, axis=-1, keepdims=True)
    return (h - m) * jax.lax.rsqrt(v + 1e-5) * g + b


def _dot(a, b):
    return jnp.dot(a, b, preferred_element_type=jnp.float32)


def _sc_gather_pair(nl, col3, row3):
    _, nch, ch = col3.shape
    epw = nch * ch
    ne = NW * epw

    @functools.partial(
        pl.kernel,
        mesh=_MESH,
        out_type=[jax.ShapeDtypeStruct((ne, LAT), jnp.float32)] * 2,
        scratch_types=[
            pltpu.VMEM((nch, ch), jnp.int32),
            pltpu.VMEM((nch, ch), jnp.int32),
            pltpu.VMEM((ND, ch, LAT), jnp.float32),
            pltpu.VMEM((ND, ch, LAT), jnp.float32),
        ] + [pltpu.SemaphoreType.DMA] * (2 * ND),
    )
    def k(nl_h, col_h, row_h, outc_h, outr_h, idxc, idxr, bufc, bufr, *sems):
        wid = lax.axis_index("s") * NC + lax.axis_index("c")
        base = wid * epw
        pltpu.sync_copy(col_h.at[wid], idxc)
        pltpu.sync_copy(row_h.at[wid], idxr)
        csems = sems[:ND]
        rsems = sems[ND:]

        def start(j, b):
            pltpu.async_copy(nl_h.at[idxc.at[j]], bufc.at[b], csems[b])
            pltpu.async_copy(nl_h.at[idxr.at[j]], bufr.at[b], rsems[b])

        def finish(j, b):
            pltpu.make_async_copy(nl_h.at[idxc.at[j]], bufc.at[b],
                                  csems[b]).wait()
            pltpu.make_async_copy(nl_h.at[idxr.at[j]], bufr.at[b],
                                  rsems[b]).wait()
            pltpu.sync_copy(bufc.at[b], outc_h.at[pl.ds(base + j * ch, ch)])
            pltpu.sync_copy(bufr.at[b], outr_h.at[pl.ds(base + j * ch, ch)])

        for b in range(ND):
            start(b, b)

        def body(i, carry):
            j0 = i * ND
            for b in range(ND):
                j = j0 + b
                finish(j, b)

                @pl.when(j + ND < nch)
                def _():
                    start(j + ND, b)
            return carry

        lax.fori_loop(0, nch // ND, body, 0)
        for j in range((nch // ND) * ND, nch):
            finish(j, j % ND)

    return k(nl, col3, row3)


def _sc_scatter(msg, col3, zrows):
    _, nch, ch = col3.shape
    epw = nch * ch

    @functools.partial(
        pl.kernel,
        mesh=_MESH,
        out_type=jax.ShapeDtypeStruct((NC, NNODE, LAT), jnp.float32),
        scratch_types=[
            pltpu.VMEM((nch, ch), jnp.int32),
            pltpu.VMEM((NDS, ch, LAT), jnp.float32),
            pltpu.VMEM_SHARED((NNODE, LAT), jnp.float32),
        ] + [pltpu.SemaphoreType.DMA] * NDS,
    )
    def k(msg_h, col_h, z_h, out_h, idxc, mbuf, acc, *sems):
        sid = lax.axis_index("s")
        cid = lax.axis_index("c")
        wid = sid * NC + cid
        base = wid * epw
        pltpu.sync_copy(col_h.at[wid], idxc)

        nzch = NNODE // CH

        def zbody(t, carry):
            j = t * 16 + sid

            @pl.when(j < nzch)
            def _():
                pltpu.sync_copy(z_h, acc.at[pl.ds(j * CH, CH)])
            return carry

        lax.fori_loop(0, (nzch + 15) // 16, zbody, 0)
        plsc.subcore_barrier()

        def load(j, b):
            pltpu.async_copy(msg_h.at[pl.ds(base + j * ch, ch)], mbuf.at[b],
                             sems[b])

        def finish(j, b):
            pltpu.make_async_copy(msg_h.at[pl.ds(base + j * ch, ch)],
                                  mbuf.at[b], sems[b]).wait()
            pltpu.sync_copy(mbuf.at[b], acc.at[idxc.at[j]], add=True)

        for b in range(NDS):
            load(b, b)

        def body(i, carry):
            j0 = i * NDS
            for b in range(NDS):
                j = j0 + b
                finish(j, b)

                @pl.when(j + NDS < nch)
                def _():
                    load(j + NDS, b)
            return carry

        lax.fori_loop(0, nch // NDS, body, 0)
        for j in range((nch // NDS) * NDS, nch):
            finish(j, j % NDS)
        plsc.subcore_barrier()

        def wbody(t, carry):
            j = t * 16 + sid

            @pl.when(j < nzch)
            def _():
                pltpu.sync_copy(acc.at[pl.ds(j * CH, CH)],
                                out_h.at[cid].at[pl.ds(j * CH, CH)])
            return carry

        lax.fori_loop(0, (nzch + 15) // 16, wbody, 0)

    return k(msg, col3, zrows)


def _encode_nodes_body(x, neW1t, neb1, neW2t, neb2, neg, nebe, nl_o):
    h = jax.nn.relu(_dot(x[...], neW1t[...]) + neb1[...])
    h = jax.nn.relu(_dot(h, neW2t[...]) + neb2[...])
    nl_o[...] = _ln(h, neg[...], nebe[...])


def _encode_nodes(x, neW1t, neb1, neW2t, neb2, neg, nebe):
    return pl.pallas_call(
        _encode_nodes_body,
        out_shape=jax.ShapeDtypeStruct((NNODE, LAT), jnp.float32),
    )(x, neW1t, neb1, neW2t, neb2, neg, nebe)


def _edge1_body(ea, nlc, nlr, eeW1t, eeb1, eeW2t, eeb2, eeg, eebe,
                w1a, w1b, w1ct, b1, w2t, b2, gg, gbe, msg_o, el_o):
    h = jax.nn.relu(_dot(ea[...], eeW1t[...]) + eeb1[...])
    h = jax.nn.relu(_dot(h, eeW2t[...]) + eeb2[...])
    el = _ln(h, eeg[...], eebe[...])
    ce = _dot(el, w1ct[...]) + b1[...]
    tc = _dot(nlc[...], w1a[...])
    tr = _dot(nlr[...], w1b[...])
    m = jax.nn.relu(tc + tr + ce)
    m = jax.nn.relu(_dot(m, w2t[...]) + b2[...])
    msg_o[...] = _ln(m, gg[...], gbe[...])
    n = jax.nn.relu(_dot(nlr[...], w1a[...]) + _dot(nlc[...], w1b[...]) + ce)
    n = jax.nn.relu(_dot(n, w2t[...]) + b2[...])
    el_o[...] = _ln(n, gg[...], gbe[...]) + el


def _edge_step1(ea, nlc, nlr, eeW1t, eeb1, eeW2t, eeb2, eeg, eebe,
                w1a, w1b, w1ct, b1, w2t, b2, gg, gbe):
    ne = ea.shape[0]
    nblk = ne // EB
    eb_spec = pl.BlockSpec((EB, LAT), lambda i: (i, 0))
    ea_spec = pl.BlockSpec((EB, 16), lambda i: (i, 0))
    w_spec = lambda s: pl.BlockSpec(s, lambda i: (0,) * len(s))
    return pl.pallas_call(
        _edge1_body,
        grid=(nblk,),
        in_specs=[ea_spec, eb_spec, eb_spec,
                  w_spec((16, LAT)), w_spec((1, LAT)), w_spec((LAT, LAT)),
                  w_spec((1, LAT)), w_spec((1, LAT)), w_spec((1, LAT)),
                  w_spec((LAT, LAT)), w_spec((LAT, LAT)), w_spec((LAT, LAT)),
                  w_spec((1, LAT)), w_spec((LAT, LAT)), w_spec((1, LAT)),
                  w_spec((1, LAT)), w_spec((1, LAT))],
        out_specs=[eb_spec, eb_spec],
        out_shape=[jax.ShapeDtypeStruct((ne, LAT), jnp.float32)] * 2,
    )(ea, nlc, nlr, eeW1t, eeb1, eeW2t, eeb2, eeg, eebe,
      w1a, w1b, w1ct, b1, w2t, b2, gg, gbe)


def _edge2_body(el, nlc, nlr, w1a, w1b, w1ct, b1, w2t, b2, gg, gbe, msg_o):
    ce = _dot(el[...], w1ct[...]) + b1[...]
    m = jax.nn.relu(_dot(nlc[...], w1a[...]) + _dot(nlr[...], w1b[...]) + ce)
    m = jax.nn.relu(_dot(m, w2t[...]) + b2[...])
    msg_o[...] = _ln(m, gg[...], gbe[...])


def _edge_step2(el, nlc, nlr, w1a, w1b, w1ct, b1, w2t, b2, gg, gbe):
    ne = el.shape[0]
    nblk = ne // EB
    eb_spec = pl.BlockSpec((EB, LAT), lambda i: (i, 0))
    w_spec = lambda s: pl.BlockSpec(s, lambda i: (0,) * len(s))
    return pl.pallas_call(
        _edge2_body,
        grid=(nblk,),
        in_specs=[eb_spec, eb_spec, eb_spec,
                  w_spec((LAT, LAT)), w_spec((LAT, LAT)), w_spec((LAT, LAT)),
                  w_spec((1, LAT)), w_spec((LAT, LAT)), w_spec((1, LAT)),
                  w_spec((1, LAT)), w_spec((1, LAT))],
        out_specs=eb_spec,
        out_shape=jax.ShapeDtypeStruct((ne, LAT), jnp.float32),
    )(el, nlc, nlr, w1a, w1b, w1ct, b1, w2t, b2, gg, gbe)


def _node_upd_body(ps, nl, wat, wbt, b1, w2t, b2, gg, gbe, nl_o):
    aggr = ps[0]
    for i in range(1, ps.shape[0]):
        aggr = aggr + ps[i]
    h = jax.nn.relu(_dot(aggr, wat[...]) + _dot(nl[...], wbt[...]) + b1[...])
    h = jax.nn.relu(_dot(h, w2t[...]) + b2[...])
    nl_o[...] = _ln(h, gg[...], gbe[...]) + nl[...]


def _node_update(ps, nl, wat, wbt, b1, w2t, b2, gg, gbe):
    return pl.pallas_call(
        _node_upd_body,
        out_shape=jax.ShapeDtypeStruct((NNODE, LAT), jnp.float32),
    )(ps, nl, wat, wbt, b1, w2t, b2, gg, gbe)


def _decode_body(ps, nl, wat, wbt, b1, w2t, b2, gg, gbe,
                 g1wt, g1bi, g1hr, g1hz, g1hn,
                 g2wt, g2bi, g2hr, g2hz, g2hn,
                 dW1t, db1, dW2t, db2, out_o):
    aggr = ps[0]
    for i in range(1, ps.shape[0]):
        aggr = aggr + ps[i]
    h = jax.nn.relu(_dot(aggr, wat[...]) + _dot(nl[...], wbt[...]) + b1[...])
    h = jax.nn.relu(_dot(h, w2t[...]) + b2[...])
    nl2 = _ln(h, gg[...], gbe[...]) + nl[...]

    gi = _dot(nl2, g1wt[...]) + g1bi[...]
    r = jax.nn.sigmoid(gi[:, :LAT] + g1hr[...])
    z = jax.nn.sigmoid(gi[:, LAT:2 * LAT] + g1hz[...])
    nn = jnp.tanh(gi[:, 2 * LAT:] + r * g1hn[...])
    h1 = (1.0 - z) * nn

    gi2 = _dot(h1, g2wt[...]) + g2bi[...]
    r2 = jax.nn.sigmoid(gi2[:, :LAT] + g2hr[...])
    z2 = jax.nn.sigmoid(gi2[:, LAT:2 * LAT] + g2hz[...])
    nn2 = jnp.tanh(gi2[:, 2 * LAT:] + r2 * g2hn[...])
    h2 = (1.0 - z2) * nn2

    d = jax.nn.relu(_dot(h2, dW1t[...]) + db1[...])
    out_o[...] = _dot(d, dW2t[...]) + db2[...]


def _decode(ps, nl, wat, wbt, b1, w2t, b2, gg, gbe,
            g1wt, g1bi, g1hr, g1hz, g1hn, g2wt, g2bi, g2hr, g2hz, g2hn,
            dW1t, db1, dW2t, db2):
    return pl.pallas_call(
        _decode_body,
        out_shape=jax.ShapeDtypeStruct((NNODE, LAT), jnp.float32),
    )(ps, nl, wat, wbt, b1, w2t, b2, gg, gbe,
      g1wt, g1bi, g1hr, g1hz, g1hn, g2wt, g2bi, g2hr, g2hz, g2hn,
      dW1t, db1, dW2t, db2)


def kernel(x, edge_index, edge_attr, ne_W1, ne_b1, ne_W2, ne_b2, ne_g, ne_be,
           ee_W1, ee_b1, ee_W2, ee_b2, ee_g, ee_be,
           gbe_W1, gbe_b1, gbe_W2, gbe_b2, gbe_g, gbe_be,
           gbn_W1, gbn_b1, gbn_W2, gbn_b2, gbn_g, gbn_be,
           g1_Wih, g1_Whh, g1_bih, g1_bhh,
           g2_Wih, g2_Whh, g2_bih, g2_bhh,
           dec_W1, dec_b1, dec_W2, dec_b2):
    eq = NEDGE // Q
    nchq = eq // (NW * CH)
    row4 = edge_index[0].reshape(Q, NW, nchq, CH)
    col4 = edge_index[1].reshape(Q, NW, nchq, CH)
    ea3 = edge_attr.reshape(Q, eq, 16)
    r2 = lambda v: v.reshape(1, -1)
    zrows = jnp.zeros((CH, LAT), jnp.float32)

    w1at = gbe_W1[:, :LAT].T
    w1bt = gbe_W1[:, LAT:2 * LAT].T
    w1ct = gbe_W1[:, 2 * LAT:].T
    w2te = gbe_W2.T
    wat = gbn_W1[:, :LAT].T
    wbt = gbn_W1[:, LAT:].T

    nl = _encode_nodes(x, ne_W1.T, r2(ne_b1), ne_W2.T, r2(ne_b2),
                       r2(ne_g), r2(ne_be))

    gpairs = [_sc_gather_pair(nl, col4[q], row4[q]) for q in range(Q)]
    msgs, els = [], []
    for q in range(Q):
        nlc, nlr = gpairs[q]
        m_q, el_q = _edge_step1(ea3[q], nlc, nlr, ee_W1.T, r2(ee_b1),
                                ee_W2.T, r2(ee_b2), r2(ee_g), r2(ee_be),
                                w1at, w1bt, w1ct, r2(gbe_b1), w2te,
                                r2(gbe_b2), r2(gbe_g), r2(gbe_be))
        msgs.append(m_q)
        els.append(el_q)
    parts = [_sc_scatter(msgs[q], col4[q], zrows) for q in range(Q)]
    ps = parts[0] if Q == 1 else jnp.concatenate(parts, axis=0)
    nl = _node_update(ps, nl, wat, wbt, r2(gbn_b1), gbn_W2.T,
                      r2(gbn_b2), r2(gbn_g), r2(gbn_be))

    gpairs = [_sc_gather_pair(nl, col4[q], row4[q]) for q in range(Q)]
    msgs = []
    for q in range(Q):
        nlc, nlr = gpairs[q]
        msgs.append(_edge_step2(els[q], nlc, nlr, w1at, w1bt, w1ct,
                                r2(gbe_b1), w2te, r2(gbe_b2), r2(gbe_g),
                                r2(gbe_be)))
    parts = [_sc_scatter(msgs[q], col4[q], zrows) for q in range(Q)]
    ps = parts[0] if Q == 1 else jnp.concatenate(parts, axis=0)

    g1hr, g1hz, g1hn = g1_bhh[:LAT], g1_bhh[LAT:2 * LAT], g1_bhh[2 * LAT:]
    g2hr, g2hz, g2hn = g2_bhh[:LAT], g2_bhh[LAT:2 * LAT], g2_bhh[2 * LAT:]
    dW2t = jnp.zeros((LAT, LAT), jnp.float32).at[:, :3].set(dec_W2.T)
    db2 = jnp.zeros((1, LAT), jnp.float32).at[:, :3].set(dec_b2)
    dec = _decode(ps, nl, wat, wbt, r2(gbn_b1), gbn_W2.T, r2(gbn_b2),
                  r2(gbn_g), r2(gbn_be),
                  g1_Wih.T, r2(g1_bih), r2(g1hr), r2(g1hz), r2(g1hn),
                  g2_Wih.T, r2(g2_bih), r2(g2hr), r2(g2hz), r2(g2hn),
                  dec_W1.T, r2(dec_b1), dW2t, db2)
    return dec[None, :, :3]


# speedup vs baseline: 1.1117x; 1.0010x over previous
import functools

import jax
import jax.numpy as jnp
from jax import lax
from jax.experimental import pallas as pl
from jax.experimental.pallas import tpu as pltpu
from jax.experimental.pallas import tpu_sc as plsc

LAT = 128
NNODE = 10000
NEDGE = 320000
EB = 8000
NC = 2
NW = 32
CH = 80
Q = 1
ND = 4
NDS = 3

_MESH = plsc.VectorSubcoreMesh(core_axis_name="c", subcore_axis_name="s")


def _ln(h, g, b):
    m = jnp.mean(h, axis=-1, keepdims=True)
    v = jnp.mean((h - m) ** 2, axis=-1, keepdims=True)
    return (h - m) * jax.lax.rsqrt(v + 1e-5) * g + b


def _dot(a, b):
    return jnp.dot(a, b, preferred_element_type=jnp.float32)


def _sc_gather_pair(nl, col3, row3):
    _, nch, ch = col3.shape
    epw = nch * ch
    ne = NW * epw

    @functools.partial(
        pl.kernel,
        mesh=_MESH,
        out_type=[jax.ShapeDtypeStruct((ne, LAT), jnp.float32)] * 2,
        scratch_types=[
            pltpu.VMEM((nch, ch), jnp.int32),
            pltpu.VMEM((nch, ch), jnp.int32),
            pltpu.VMEM((ND, ch, LAT), jnp.float32),
            pltpu.VMEM((ND, ch, LAT), jnp.float32),
        ] + [pltpu.SemaphoreType.DMA] * (2 * ND),
    )
    def k(nl_h, col_h, row_h, outc_h, outr_h, idxc, idxr, bufc, bufr, *sems):
        wid = lax.axis_index("s") * NC + lax.axis_index("c")
        base = wid * epw
        pltpu.sync_copy(col_h.at[wid], idxc)
        pltpu.sync_copy(row_h.at[wid], idxr)
        csems = sems[:ND]
        rsems = sems[ND:]

        def start(j, b):
            pltpu.async_copy(nl_h.at[idxc.at[j]], bufc.at[b], csems[b])
            pltpu.async_copy(nl_h.at[idxr.at[j]], bufr.at[b], rsems[b])

        def finish(j, b):
            pltpu.make_async_copy(nl_h.at[idxc.at[j]], bufc.at[b],
                                  csems[b]).wait()
            pltpu.make_async_copy(nl_h.at[idxr.at[j]], bufr.at[b],
                                  rsems[b]).wait()
            pltpu.sync_copy(bufc.at[b], outc_h.at[pl.ds(base + j * ch, ch)])
            pltpu.sync_copy(bufr.at[b], outr_h.at[pl.ds(base + j * ch, ch)])

        for b in range(ND):
            start(b, b)

        def body(i, carry):
            j0 = i * ND
            for b in range(ND):
                j = j0 + b
                finish(j, b)

                @pl.when(j + ND < nch)
                def _():
                    start(j + ND, b)
            return carry

        lax.fori_loop(0, nch // ND, body, 0)
        for j in range((nch // ND) * ND, nch):
            finish(j, j % ND)

    return k(nl, col3, row3)


def _sc_scatter(msg, col3, zrows):
    _, nch, ch = col3.shape
    epw = nch * ch

    @functools.partial(
        pl.kernel,
        mesh=_MESH,
        out_type=jax.ShapeDtypeStruct((NC, NNODE, LAT), jnp.float32),
        scratch_types=[
            pltpu.VMEM((nch, ch), jnp.int32),
            pltpu.VMEM((NDS, ch, LAT), jnp.float32),
            pltpu.VMEM_SHARED((NNODE, LAT), jnp.float32),
        ] + [pltpu.SemaphoreType.DMA] * NDS,
    )
    def k(msg_h, col_h, z_h, out_h, idxc, mbuf, acc, *sems):
        sid = lax.axis_index("s")
        cid = lax.axis_index("c")
        wid = sid * NC + cid
        base = wid * epw
        pltpu.sync_copy(col_h.at[wid], idxc)

        nzch = NNODE // CH

        def zbody(t, carry):
            j = t * 16 + sid

            @pl.when(j < nzch)
            def _():
                pltpu.sync_copy(z_h, acc.at[pl.ds(j * CH, CH)])
            return carry

        lax.fori_loop(0, (nzch + 15) // 16, zbody, 0)
        plsc.subcore_barrier()

        def load(j, b):
            pltpu.async_copy(msg_h.at[pl.ds(base + j * ch, ch)], mbuf.at[b],
                             sems[b])

        def finish(j, b):
            pltpu.make_async_copy(msg_h.at[pl.ds(base + j * ch, ch)],
                                  mbuf.at[b], sems[b]).wait()
            pltpu.sync_copy(mbuf.at[b], acc.at[idxc.at[j]], add=True)

        for b in range(NDS):
            load(b, b)

        def body(i, carry):
            j0 = i * NDS
            for b in range(NDS):
                j = j0 + b
                finish(j, b)

                @pl.when(j + NDS < nch)
                def _():
                    load(j + NDS, b)
            return carry

        lax.fori_loop(0, nch // NDS, body, 0)
        for j in range((nch // NDS) * NDS, nch):
            finish(j, j % NDS)
        plsc.subcore_barrier()

        def wbody(t, carry):
            j = t * 16 + sid

            @pl.when(j < nzch)
            def _():
                pltpu.sync_copy(acc.at[pl.ds(j * CH, CH)],
                                out_h.at[cid].at[pl.ds(j * CH, CH)])
            return carry

        lax.fori_loop(0, (nzch + 15) // 16, wbody, 0)

    return k(msg, col3, zrows)


def _encode_nodes_body(x, neW1t, neb1, neW2t, neb2, neg, nebe, nl_o):
    h = jax.nn.relu(_dot(x[...], neW1t[...]) + neb1[...])
    h = jax.nn.relu(_dot(h, neW2t[...]) + neb2[...])
    nl_o[...] = _ln(h, neg[...], nebe[...])


def _encode_nodes(x, neW1t, neb1, neW2t, neb2, neg, nebe):
    return pl.pallas_call(
        _encode_nodes_body,
        out_shape=jax.ShapeDtypeStruct((NNODE, LAT), jnp.float32),
    )(x, neW1t, neb1, neW2t, neb2, neg, nebe)


def _edge1_body(ea, nlc, nlr, eeW1t, eeb1, eeW2t, eeb2, eeg, eebe,
                w1a, w1b, w1ct, b1, w2t, b2, gg, gbe, msg_o, el_o):
    h = jax.nn.relu(_dot(ea[...], eeW1t[...]) + eeb1[...])
    h = jax.nn.relu(_dot(h, eeW2t[...]) + eeb2[...])
    el = _ln(h, eeg[...], eebe[...])
    ce = _dot(el, w1ct[...]) + b1[...]
    tc = _dot(nlc[...], w1a[...])
    tr = _dot(nlr[...], w1b[...])
    m = jax.nn.relu(tc + tr + ce)
    m = jax.nn.relu(_dot(m, w2t[...]) + b2[...])
    msg_o[...] = _ln(m, gg[...], gbe[...])
    n = jax.nn.relu(_dot(nlr[...], w1a[...]) + _dot(nlc[...], w1b[...]) + ce)
    n = jax.nn.relu(_dot(n, w2t[...]) + b2[...])
    el_o[...] = _ln(n, gg[...], gbe[...]) + el


def _edge_step1(ea, nlc, nlr, eeW1t, eeb1, eeW2t, eeb2, eeg, eebe,
                w1a, w1b, w1ct, b1, w2t, b2, gg, gbe):
    ne = ea.shape[0]
    nblk = ne // EB
    eb_spec = pl.BlockSpec((EB, LAT), lambda i: (i, 0))
    ea_spec = pl.BlockSpec((EB, 16), lambda i: (i, 0))
    w_spec = lambda s: pl.BlockSpec(s, lambda i: (0,) * len(s))
    return pl.pallas_call(
        _edge1_body,
        grid=(nblk,),
        in_specs=[ea_spec, eb_spec, eb_spec,
                  w_spec((16, LAT)), w_spec((1, LAT)), w_spec((LAT, LAT)),
                  w_spec((1, LAT)), w_spec((1, LAT)), w_spec((1, LAT)),
                  w_spec((LAT, LAT)), w_spec((LAT, LAT)), w_spec((LAT, LAT)),
                  w_spec((1, LAT)), w_spec((LAT, LAT)), w_spec((1, LAT)),
                  w_spec((1, LAT)), w_spec((1, LAT))],
        out_specs=[eb_spec, eb_spec],
        out_shape=[jax.ShapeDtypeStruct((ne, LAT), jnp.float32)] * 2,
    )(ea, nlc, nlr, eeW1t, eeb1, eeW2t, eeb2, eeg, eebe,
      w1a, w1b, w1ct, b1, w2t, b2, gg, gbe)


def _edge2_body(el, nlc, nlr, w1a, w1b, w1ct, b1, w2t, b2, gg, gbe, msg_o):
    ce = _dot(el[...], w1ct[...]) + b1[...]
    m = jax.nn.relu(_dot(nlc[...], w1a[...]) + _dot(nlr[...], w1b[...]) + ce)
    m = jax.nn.relu(_dot(m, w2t[...]) + b2[...])
    msg_o[...] = _ln(m, gg[...], gbe[...])


def _edge_step2(el, nlc, nlr, w1a, w1b, w1ct, b1, w2t, b2, gg, gbe):
    ne = el.shape[0]
    nblk = ne // EB
    eb_spec = pl.BlockSpec((EB, LAT), lambda i: (i, 0))
    w_spec = lambda s: pl.BlockSpec(s, lambda i: (0,) * len(s))
    return pl.pallas_call(
        _edge2_body,
        grid=(nblk,),
        in_specs=[eb_spec, eb_spec, eb_spec,
                  w_spec((LAT, LAT)), w_spec((LAT, LAT)), w_spec((LAT, LAT)),
                  w_spec((1, LAT)), w_spec((LAT, LAT)), w_spec((1, LAT)),
                  w_spec((1, LAT)), w_spec((1, LAT))],
        out_specs=eb_spec,
        out_shape=jax.ShapeDtypeStruct((ne, LAT), jnp.float32),
    )(el, nlc, nlr, w1a, w1b, w1ct, b1, w2t, b2, gg, gbe)


def _node_upd_body(ps, nl, wat, wbt, b1, w2t, b2, gg, gbe, nl_o):
    aggr = ps[0]
    for i in range(1, ps.shape[0]):
        aggr = aggr + ps[i]
    h = jax.nn.relu(_dot(aggr, wat[...]) + _dot(nl[...], wbt[...]) + b1[...])
    h = jax.nn.relu(_dot(h, w2t[...]) + b2[...])
    nl_o[...] = _ln(h, gg[...], gbe[...]) + nl[...]


def _node_update(ps, nl, wat, wbt, b1, w2t, b2, gg, gbe):
    return pl.pallas_call(
        _node_upd_body,
        out_shape=jax.ShapeDtypeStruct((NNODE, LAT), jnp.float32),
    )(ps, nl, wat, wbt, b1, w2t, b2, gg, gbe)


def _decode_body(ps, nl, wat, wbt, b1, w2t, b2, gg, gbe,
                 g1wt, g1bi, g1hr, g1hz, g1hn,
                 g2wt, g2bi, g2hr, g2hz, g2hn,
                 dW1t, db1, dW2t, db2, out_o):
    aggr = ps[0]
    for i in range(1, ps.shape[0]):
        aggr = aggr + ps[i]
    h = jax.nn.relu(_dot(aggr, wat[...]) + _dot(nl[...], wbt[...]) + b1[...])
    h = jax.nn.relu(_dot(h, w2t[...]) + b2[...])
    nl2 = _ln(h, gg[...], gbe[...]) + nl[...]

    gi = _dot(nl2, g1wt[...]) + g1bi[...]
    r = jax.nn.sigmoid(gi[:, :LAT] + g1hr[...])
    z = jax.nn.sigmoid(gi[:, LAT:2 * LAT] + g1hz[...])
    nn = jnp.tanh(gi[:, 2 * LAT:] + r * g1hn[...])
    h1 = (1.0 - z) * nn

    gi2 = _dot(h1, g2wt[...]) + g2bi[...]
    r2 = jax.nn.sigmoid(gi2[:, :LAT] + g2hr[...])
    z2 = jax.nn.sigmoid(gi2[:, LAT:2 * LAT] + g2hz[...])
    nn2 = jnp.tanh(gi2[:, 2 * LAT:] + r2 * g2hn[...])
    h2 = (1.0 - z2) * nn2

    d = jax.nn.relu(_dot(h2, dW1t[...]) + db1[...])
    out_o[...] = _dot(d, dW2t[...]) + db2[...]


def _decode(ps, nl, wat, wbt, b1, w2t, b2, gg, gbe,
            g1wt, g1bi, g1hr, g1hz, g1hn, g2wt, g2bi, g2hr, g2hz, g2hn,
            dW1t, db1, dW2t, db2):
    return pl.pallas_call(
        _decode_body,
        out_shape=jax.ShapeDtypeStruct((NNODE, LAT), jnp.float32),
    )(ps, nl, wat, wbt, b1, w2t, b2, gg, gbe,
      g1wt, g1bi, g1hr, g1hz, g1hn, g2wt, g2bi, g2hr, g2hz, g2hn,
      dW1t, db1, dW2t, db2)


def kernel(x, edge_index, edge_attr, ne_W1, ne_b1, ne_W2, ne_b2, ne_g, ne_be,
           ee_W1, ee_b1, ee_W2, ee_b2, ee_g, ee_be,
           gbe_W1, gbe_b1, gbe_W2, gbe_b2, gbe_g, gbe_be,
           gbn_W1, gbn_b1, gbn_W2, gbn_b2, gbn_g, gbn_be,
           g1_Wih, g1_Whh, g1_bih, g1_bhh,
           g2_Wih, g2_Whh, g2_bih, g2_bhh,
           dec_W1, dec_b1, dec_W2, dec_b2):
    eq = NEDGE // Q
    nchq = eq // (NW * CH)
    row4 = edge_index[0].reshape(Q, NW, nchq, CH)
    col4 = edge_index[1].reshape(Q, NW, nchq, CH)
    ea3 = edge_attr.reshape(Q, eq, 16)
    r2 = lambda v: v.reshape(1, -1)
    zrows = jnp.zeros((CH, LAT), jnp.float32)

    w1at = gbe_W1[:, :LAT].T
    w1bt = gbe_W1[:, LAT:2 * LAT].T
    w1ct = gbe_W1[:, 2 * LAT:].T
    w2te = gbe_W2.T
    wat = gbn_W1[:, :LAT].T
    wbt = gbn_W1[:, LAT:].T

    nl = _encode_nodes(x, ne_W1.T, r2(ne_b1), ne_W2.T, r2(ne_b2),
                       r2(ne_g), r2(ne_be))

    gpairs = [_sc_gather_pair(nl, col4[q], row4[q]) for q in range(Q)]
    msgs, els = [], []
    for q in range(Q):
        nlc, nlr = gpairs[q]
        m_q, el_q = _edge_step1(ea3[q], nlc, nlr, ee_W1.T, r2(ee_b1),
                                ee_W2.T, r2(ee_b2), r2(ee_g), r2(ee_be),
                                w1at, w1bt, w1ct, r2(gbe_b1), w2te,
                                r2(gbe_b2), r2(gbe_g), r2(gbe_be))
        msgs.append(m_q)
        els.append(el_q)
    parts = [_sc_scatter(msgs[q], col4[q], zrows) for q in range(Q)]
    ps = parts[0] if Q == 1 else jnp.concatenate(parts, axis=0)
    nl = _node_update(ps, nl, wat, wbt, r2(gbn_b1), gbn_W2.T,
                      r2(gbn_b2), r2(gbn_g), r2(gbn_be))

    gpairs = [_sc_gather_pair(nl, col4[q], row4[q]) for q in range(Q)]
    msgs = []
    for q in range(Q):
        nlc, nlr = gpairs[q]
        msgs.append(_edge_step2(els[q], nlc, nlr, w1at, w1bt, w1ct,
                                r2(gbe_b1), w2te, r2(gbe_b2), r2(gbe_g),
                                r2(gbe_be)))
    parts = [_sc_scatter(msgs[q], col4[q], zrows) for q in range(Q)]
    ps = parts[0] if Q == 1 else jnp.concatenate(parts, axis=0)

    g1hr, g1hz, g1hn = g1_bhh[:LAT], g1_bhh[LAT:2 * LAT], g1_bhh[2 * LAT:]
    g2hr, g2hz, g2hn = g2_bhh[:LAT], g2_bhh[LAT:2 * LAT], g2_bhh[2 * LAT:]
    dW2t = jnp.zeros((LAT, LAT), jnp.float32).at[:, :3].set(dec_W2.T)
    db2 = jnp.zeros((1, LAT), jnp.float32).at[:, :3].set(dec_b2)
    dec = _decode(ps, nl, wat, wbt, r2(gbn_b1), gbn_W2.T, r2(gbn_b2),
                  r2(gbn_g), r2(gbn_be),
                  g1_Wih.T, r2(g1_bih), r2(g1hr), r2(g1hz), r2(g1hn),
                  g2_Wih.T, r2(g2_bih), r2(g2hr), r2(g2hz), r2(g2hn),
                  dec_W1.T, r2(dec_b1), dW2t, db2)
    return dec[None, :, :3]
